# baseline (speedup 1.0000x reference)
import functools

import jax
import jax.numpy as jnp
from jax import lax
from jax.experimental import pallas as pl
from jax.experimental.pallas import tpu as pltpu
from jax.experimental.pallas import tpu_sc as plsc

N_ENT = 10000
N_PAD = 10240
N_DIM = 64
N_LAYER = 2
N_RELP = 32
PAD_REL = 29
BN = 2
BD = BN * N_DIM
E_EDGES = 160000
CH = 64
N_WORKERS = 32
CPT = 80
E_PADDED = N_WORKERS * CPT * CH
ROWS_PER_TILE = N_PAD // 16
ROW_BLK = 2048


def _sc_mesh():
  return plsc.VectorSubcoreMesh(core_axis_name="c", subcore_axis_name="s",
                                num_cores=2, num_subcores=16)


EPT = CPT * CH


def _rspmm(src, typ_a, typ_b, dst, h_a, h_b, tab_a, tab_b, zeros):
  @functools.partial(
      pl.kernel,
      out_type=jax.ShapeDtypeStruct((4 * N_PAD, BD), jnp.float32),
      mesh=_sc_mesh(),
      scratch_types=[
          pltpu.VMEM((CH,), jnp.int32),
          pltpu.VMEM((CH,), jnp.int32),
          pltpu.VMEM((CH,), jnp.int32),
          pltpu.VMEM((CH,), jnp.int32),
          pltpu.VMEM((CH,), jnp.int32),
          pltpu.VMEM((CH,), jnp.int32),
          pltpu.VMEM((CH, BD), jnp.float32),
          pltpu.VMEM((CH, BD), jnp.float32),
          pltpu.VMEM((CH, BD), jnp.float32),
          pltpu.VMEM((CH, BD), jnp.float32),
          pltpu.SemaphoreType.DMA,
          pltpu.SemaphoreType.DMA,
          pltpu.SemaphoreType.DMA,
          pltpu.SemaphoreType.DMA,
          pltpu.SemaphoreType.DMA,
          pltpu.SemaphoreType.DMA,
          pltpu.VMEM_SHARED((N_PAD, BD), jnp.float32),
      ],
  )
  def body(src_hbm, ta_hbm, tb_hbm, dst_hbm, ha_hbm, hb_hbm, taba_hbm,
           tabb_hbm, z_hbm, out_hbm, sva, svb, tva, tvb, dva, dvb,
           hva, rva, hvb, rvb, sia, sib, sha, shb, sra, srb, agg):
    cid = lax.axis_index("c")
    sid = lax.axis_index("s")
    wid = sid * 2 + cid
    r0 = sid * ROWS_PER_TILE
    base = wid * EPT

    for phase, (t_hbm, h_hbm, tab_hbm) in enumerate(
        ((ta_hbm, ha_hbm, taba_hbm), (tb_hbm, hb_hbm, tabb_hbm))):
      pltpu.sync_copy(z_hbm.at[pl.ds(r0, ROWS_PER_TILE)],
                      agg.at[pl.ds(r0, ROWS_PER_TILE)])
      plsc.subcore_barrier()

      def fire_idx(ch, sv, tv, dv, si):
        off = base + ch * CH
        pltpu.async_copy(src_hbm.at[pl.ds(off, CH)], sv, si)
        pltpu.async_copy(t_hbm.at[pl.ds(off, CH)], tv, si)
        pltpu.async_copy(dst_hbm.at[pl.ds(off, CH)], dv, si)

      def wait_idx(ch, sv, tv, dv, si):
        off = base + ch * CH
        pltpu.make_async_copy(src_hbm.at[pl.ds(off, CH)], sv, si).wait()
        pltpu.make_async_copy(t_hbm.at[pl.ds(off, CH)], tv, si).wait()
        pltpu.make_async_copy(dst_hbm.at[pl.ds(off, CH)], dv, si).wait()

      def fire_gath(sv, tv, hv, rv, sh, sr):
        pltpu.async_copy(h_hbm.at[sv], hv, sh)
        pltpu.async_copy(tab_hbm.at[tv], rv, sr)

      def process(sv, tv, dv, hv, rv, sh, sr):
        pltpu.make_async_copy(h_hbm.at[sv], hv, sh).wait()
        pltpu.make_async_copy(tab_hbm.at[tv], rv, sr).wait()

        def mul_row(j, c2):
          for c in range(BD // 16):
            hv[j, pl.ds(c * 16, 16)] = (hv[j, pl.ds(c * 16, 16)] *
                                        rv[j, pl.ds(c * 16, 16)])
          return c2

        lax.fori_loop(0, CH, mul_row, 0, unroll=False)
        pltpu.sync_copy(hv, agg.at[dv], add=True)

      setA = (sva, tva, dva, sia)
      setB = (svb, tvb, dvb, sib)
      gatA = (hva, rva, sha, sra)
      gatB = (hvb, rvb, shb, srb)

      fire_idx(0, *setA)
      wait_idx(0, *setA)
      fire_gath(setA[0], setA[1], *gatA)
      fire_idx(1, *setB)

      def halfstep(ch, cur, gcur, nxt, gnxt):
        @pl.when(ch + 1 < CPT)
        def _():
          wait_idx(ch + 1, *nxt)
          fire_gath(nxt[0], nxt[1], *gnxt)
        process(cur[0], cur[1], cur[2], *gcur)

        @pl.when(ch + 2 < CPT)
        def _():
          fire_idx(ch + 2, *cur)

      def pair(ch, carry):
        halfstep(ch, setA, gatA, setB, gatB)
        halfstep(ch + 1, setB, gatB, setA, gatA)
        return carry

      lax.fori_loop(0, CPT // 2, lambda i, c: pair(i * 2, c), 0,
                    unroll=False)
      plsc.subcore_barrier()

      pltpu.sync_copy(
          agg.at[pl.ds(r0, ROWS_PER_TILE)],
          out_hbm.at[pl.ds(phase * 2 * N_PAD + cid * N_PAD + r0,
                           ROWS_PER_TILE)])
      plsc.subcore_barrier()

  out = body(src, typ_a, typ_b, dst, h_a, h_b, tab_a, tab_b, zeros)
  return out.reshape(2, 2, N_PAD, BD)


def _relw_body(htT_ref, w1T_ref, b1T_ref, w2T_ref, b2T_ref, emb_ref, o_ref):
  xT = lax.dot_general(w1T_ref[0], htT_ref[...], (((1,), (0,)), ((), ())),
                       preferred_element_type=jnp.float32)
  xT = jnp.maximum(xT + b1T_ref[0], 0.0)
  wT = lax.dot_general(w2T_ref[0], xT, (((1,), (0,)), ((), ())),
                       preferred_element_type=jnp.float32)
  wT = jax.nn.sigmoid(wT + b2T_ref[0])
  e = emb_ref[0]
  o_ref[0] = jnp.concatenate([wT[:, 0:1] * e, wT[:, 1:2] * e], axis=1)


def _rel_tables(htT, w1T, b1T, w2T, b2T, embp):
  return pl.pallas_call(
      _relw_body,
      grid=(N_LAYER,),
      in_specs=[
          pl.BlockSpec((BD, 8), lambda l: (0, 0)),
          pl.BlockSpec((1, 8, BD), lambda l: (l, 0, 0)),
          pl.BlockSpec((1, 8, 1), lambda l: (l, 0, 0)),
          pl.BlockSpec((1, N_RELP, 8), lambda l: (l, 0, 0)),
          pl.BlockSpec((1, N_RELP, 1), lambda l: (l, 0, 0)),
          pl.BlockSpec((1, N_RELP, N_DIM), lambda l: (l, 0, 0)),
      ],
      out_specs=pl.BlockSpec((1, N_RELP, BD), lambda l: (l, 0, 0)),
      out_shape=jax.ShapeDtypeStruct((N_LAYER, N_RELP, BD), jnp.float32),
  )(htT, w1T, b1T, w2T, b2T, embp)


def _tab_body(rel_ref, emb_ref, o_ref):
  e0 = emb_ref[0:1, :]
  e1 = emb_ref[1:2, :]
  z = jnp.zeros((1, N_DIM), jnp.float32)
  hsel = jnp.concatenate([
      jnp.concatenate([z, z], axis=1),
      jnp.concatenate([e0, z], axis=1),
      jnp.concatenate([z, e1], axis=1),
      jnp.concatenate([e0, e1], axis=1),
  ], axis=0)
  rel = rel_ref[...]
  o_ref[...] = (hsel[:, None, :] * rel[None, :, :]).reshape(4 * N_RELP, BD)


def _expand_table(rel0, emb):
  return pl.pallas_call(
      _tab_body,
      out_shape=jax.ShapeDtypeStruct((4 * N_RELP, BD), jnp.float32),
  )(rel0, emb)


def _cls_body(idx_ref, src_ref, typ_ref, o_ref):
  src = src_ref[...]
  cls = (jnp.where(src == idx_ref[0], 1, 0) +
         jnp.where(src == idx_ref[1], 2, 0))
  o_ref[...] = typ_ref[...] + N_RELP * cls


def _cls_types(idx, src_p, typ_p):
  return pl.pallas_call(
      _cls_body,
      in_specs=[
          pl.BlockSpec(memory_space=pltpu.SMEM),
          pl.BlockSpec((N_WORKERS, CPT, CH), lambda: (0, 0, 0)),
          pl.BlockSpec((N_WORKERS, CPT, CH), lambda: (0, 0, 0)),
      ],
      out_specs=pl.BlockSpec((N_WORKERS, CPT, CH), lambda: (0, 0, 0)),
      out_shape=jax.ShapeDtypeStruct((N_WORKERS, CPT, CH), jnp.int32),
  )(idx, src_p, typ_p)


def _init_body(idx_ref, emb_ref, o_ref):
  i = pl.program_id(0)
  rows = jax.lax.broadcasted_iota(jnp.int32, (ROW_BLK, 1), 0) + i * ROW_BLK
  e0 = emb_ref[0:1, :]
  e1 = emb_ref[1:2, :]
  left = jnp.where(rows == idx_ref[0], e0, 0.0)
  right = jnp.where(rows == idx_ref[1], e1, 0.0)
  o_ref[...] = jnp.concatenate([left, right], axis=1)


def _init_hidden(idx, emb):
  return pl.pallas_call(
      _init_body,
      grid=(N_PAD // ROW_BLK,),
      in_specs=[
          pl.BlockSpec(memory_space=pltpu.SMEM),
          pl.BlockSpec((BN, N_DIM), lambda i: (0, 0)),
      ],
      out_specs=pl.BlockSpec((ROW_BLK, BD), lambda i: (i, 0)),
      out_shape=jax.ShapeDtypeStruct((N_PAD, BD), jnp.float32),
  )(idx, emb)


def _lin_body(p_ref, w_ref, b_ref, o_ref):
  a = p_ref[0] + p_ref[1]
  w = w_ref[...]
  b = b_ref[...]
  x1 = lax.dot_general(a[:, :N_DIM], w, (((1,), (0,)), ((), ())),
                       preferred_element_type=jnp.float32)
  x2 = lax.dot_general(a[:, N_DIM:], w, (((1,), (0,)), ((), ())),
                       preferred_element_type=jnp.float32)
  o_ref[...] = jnp.concatenate(
      [jnp.maximum(x1 + b, 0.0), jnp.maximum(x2 + b, 0.0)], axis=1)


def _combine_lin(parts, w, b):
  return pl.pallas_call(
      _lin_body,
      grid=(N_PAD // ROW_BLK,),
      in_specs=[
          pl.BlockSpec((2, ROW_BLK, BD), lambda i: (0, i, 0)),
          pl.BlockSpec((N_DIM, N_DIM), lambda i: (0, 0)),
          pl.BlockSpec((1, N_DIM), lambda i: (0, 0)),
      ],
      out_specs=pl.BlockSpec((ROW_BLK, BD), lambda i: (i, 0)),
      out_shape=jax.ShapeDtypeStruct((N_PAD, BD), jnp.float32),
  )(parts, w, b)


def _score_body(e_ref, w_ref, b_ref, o_ref):
  o_ref[...] = lax.dot_general(e_ref[...], w_ref[...],
                               (((1,), (0,)), ((), ())),
                               preferred_element_type=jnp.float32) + b_ref[...]


def _scores(embp, wp, bp):
  return pl.pallas_call(
      _score_body,
      out_shape=jax.ShapeDtypeStruct((8, 128), jnp.float32),
  )(embp, wp, bp)


@jax.jit
def _run(head, tail, edge_index, edge_type, ent_emb, rel_embs, lin_W,
         lin_b, rel_lin_W, rel_lin_b, attn_W, attn_b, Wr_W, Wr_b):
  dst = edge_index[0].astype(jnp.int32)
  src = edge_index[1].astype(jnp.int32)
  typ = edge_type.astype(jnp.int32)

  npad = E_PADDED - E_EDGES
  src_p = jnp.concatenate([src, jnp.zeros((npad,), jnp.int32)])
  dst_p = jnp.concatenate([dst, jnp.zeros((npad,), jnp.int32)])
  typ_p = jnp.concatenate([typ, jnp.full((npad,), PAD_REL, jnp.int32)])
  src_p = src_p.reshape(N_WORKERS, CPT, CH)
  dst_p = dst_p.reshape(N_WORKERS, CPT, CH)
  typ_p = typ_p.reshape(N_WORKERS, CPT, CH)

  def take2(tbl, ii):
    return jnp.concatenate([
        lax.dynamic_slice(tbl, (ii[0], 0), (1, N_DIM)),
        lax.dynamic_slice(tbl, (ii[1], 0), (1, N_DIM))], axis=0)

  head_embed = take2(ent_emb, head)
  tail_embed = take2(ent_emb, tail)
  ht = jnp.concatenate([head_embed, tail_embed], axis=-1)

  htT = jnp.transpose(ht).reshape(BD, 2)
  htT = jnp.pad(htT, ((0, 0), (0, 6)))
  w1T = jnp.pad(jnp.transpose(rel_lin_W, (0, 2, 1)), ((0, 0), (0, 3), (0, 0)))
  b1T = jnp.pad(rel_lin_b, ((0, 0), (0, 3)))[:, :, None]
  w2T = jnp.pad(jnp.transpose(attn_W, (0, 2, 1)),
                ((0, 0), (0, N_RELP - attn_W.shape[2]), (0, 3)))
  b2T = jnp.pad(attn_b, ((0, 0), (0, N_RELP - attn_b.shape[1])))[:, :, None]
  embp = jnp.pad(rel_embs, ((0, 0), (0, N_RELP - rel_embs.shape[1]), (0, 0)))
  rel_tab = _rel_tables(htT, w1T, b1T, w2T, b2T, embp)

  zeros = jnp.zeros((N_PAD, BD), jnp.float32)
  ones = jnp.ones((N_PAD, BD), jnp.float32)

  tabs, typ0s = [], []
  for init_idx, init_emb in ((head, head_embed), (tail, tail_embed)):
    ii = init_idx.astype(jnp.int32)
    tabs.append(_expand_table(rel_tab[0], init_emb))
    typ0s.append(_cls_types(ii, src_p, typ_p))

  rel128 = jnp.pad(rel_tab, ((0, 0), (0, 3 * N_RELP), (0, 0)))
  typ_s = jnp.stack([jnp.stack([typ0s[0], typ0s[1]])] +
                    [jnp.stack([typ_p, typ_p])] * (N_LAYER - 1))
  tab_s = jnp.stack([jnp.stack([tabs[0], tabs[1]])] +
                    [jnp.stack([rel128[l], rel128[l]])
                     for l in range(1, N_LAYER)])

  def step(carry, xs):
    h_a, h_b = carry
    t2, tb2, w, b = xs
    parts = _rspmm(src_p.reshape(-1), t2[0].reshape(-1), t2[1].reshape(-1),
                   dst_p.reshape(-1), h_a, h_b, tb2[0], tb2[1], zeros)
    h_a = _combine_lin(parts[0], w, b)
    h_b = _combine_lin(parts[1], w, b)
    return (h_a, h_b), 0

  (h_a, h_b), _ = lax.scan(
      step, (ones, ones),
      (typ_s, tab_s, lin_W, lin_b.reshape(N_LAYER, 1, N_DIM)))

  ht_t = _run_pick(h_a, tail)
  hh_t = _run_pick(h_b, head)

  emb_cat = jnp.concatenate([head_embed, tail_embed, hh_t, ht_t], axis=1)
  embp8 = jnp.pad(emb_cat, ((0, 6), (0, 0)))
  wp = jnp.pad(Wr_W, ((0, 0), (0, 128 - Wr_W.shape[1])))
  bp = jnp.pad(Wr_b, (0, 128 - Wr_b.shape[0])).reshape(1, 128)
  sc = _scores(embp8, wp, bp)
  return sc[:BN, :Wr_W.shape[1]]


def _run_pick(hid, idx):
  r0 = lax.dynamic_slice(hid, (idx[0], 0), (1, N_DIM))
  r1 = lax.dynamic_slice(hid, (idx[1], N_DIM), (1, N_DIM))
  return jnp.concatenate([r0, r1], axis=0)


def kernel(head, tail, edge_index, edge_type, ent_emb, rel_embs, lin_W, lin_b,
           rel_lin_W, rel_lin_b, attn_W, attn_b, Wr_W, Wr_b):
  return _run(head, tail, edge_index, edge_type, ent_emb, rel_embs, lin_W,
              lin_b, rel_lin_W, rel_lin_b, attn_W, attn_b, Wr_W, Wr_b)


# speedup vs baseline: 1.0769x; 1.0769x over previous
import functools

import jax
import jax.numpy as jnp
from jax import lax
from jax.experimental import pallas as pl
from jax.experimental.pallas import tpu as pltpu
from jax.experimental.pallas import tpu_sc as plsc

N_ENT = 10000
N_PAD = 10240
N_DIM = 64
N_LAYER = 2
N_RELP = 32
PAD_REL = 29
BN = 2
BD = BN * N_DIM
E_EDGES = 160000
CH = 80
N_WORKERS = 32
CPT = 64
E_PADDED = N_WORKERS * CPT * CH
ROWS_PER_TILE = N_PAD // 16
ROW_BLK = 2048


def _sc_mesh():
  return plsc.VectorSubcoreMesh(core_axis_name="c", subcore_axis_name="s",
                                num_cores=2, num_subcores=16)


EPT = CPT * CH


def _rspmm(src, typ, dst, h, rel, zeros):
  @functools.partial(
      pl.kernel,
      out_type=jax.ShapeDtypeStruct((2 * N_PAD, BD), jnp.float32),
      mesh=_sc_mesh(),
      scratch_types=[
          pltpu.VMEM((CH,), jnp.int32),
          pltpu.VMEM((CH,), jnp.int32),
          pltpu.VMEM((CH,), jnp.int32),
          pltpu.VMEM((CH,), jnp.int32),
          pltpu.VMEM((CH,), jnp.int32),
          pltpu.VMEM((CH,), jnp.int32),
          pltpu.VMEM((CH, BD), jnp.float32),
          pltpu.VMEM((CH, BD), jnp.float32),
          pltpu.VMEM((CH, BD), jnp.float32),
          pltpu.VMEM((CH, BD), jnp.float32),
          pltpu.SemaphoreType.DMA,
          pltpu.SemaphoreType.DMA,
          pltpu.SemaphoreType.DMA,
          pltpu.SemaphoreType.DMA,
          pltpu.SemaphoreType.DMA,
          pltpu.SemaphoreType.DMA,
          pltpu.VMEM_SHARED((N_PAD, BD), jnp.float32),
      ],
  )
  def body(src_hbm, t_hbm, dst_hbm, h_hbm, tab_hbm, z_hbm, out_hbm,
           sva, svb, tva, tvb, dva, dvb, hva, rva, hvb, rvb,
           sia, sib, sha, shb, sra, srb, agg):
    cid = lax.axis_index("c")
    sid = lax.axis_index("s")
    wid = sid * 2 + cid
    r0 = sid * ROWS_PER_TILE
    base = wid * EPT

    pltpu.sync_copy(z_hbm.at[pl.ds(r0, ROWS_PER_TILE)],
                    agg.at[pl.ds(r0, ROWS_PER_TILE)])
    plsc.subcore_barrier()

    def fire_idx(ch, sv, tv, dv, si):
      off = base + ch * CH
      pltpu.async_copy(src_hbm.at[pl.ds(off, CH)], sv, si)
      pltpu.async_copy(t_hbm.at[pl.ds(off, CH)], tv, si)
      pltpu.async_copy(dst_hbm.at[pl.ds(off, CH)], dv, si)

    def wait_idx(ch, sv, tv, dv, si):
      off = base + ch * CH
      pltpu.make_async_copy(src_hbm.at[pl.ds(off, CH)], sv, si).wait()
      pltpu.make_async_copy(t_hbm.at[pl.ds(off, CH)], tv, si).wait()
      pltpu.make_async_copy(dst_hbm.at[pl.ds(off, CH)], dv, si).wait()

    def fire_gath(sv, tv, hv, rv, sh, sr):
      pltpu.async_copy(h_hbm.at[sv], hv, sh)
      pltpu.async_copy(tab_hbm.at[tv], rv, sr)

    def process(sv, tv, dv, hv, rv, sh, sr):
      pltpu.make_async_copy(h_hbm.at[sv], hv, sh).wait()
      pltpu.make_async_copy(tab_hbm.at[tv], rv, sr).wait()

      def mul_row(j, c2):
        for c in range(BD // 16):
          hv[j, pl.ds(c * 16, 16)] = (hv[j, pl.ds(c * 16, 16)] *
                                      rv[j, pl.ds(c * 16, 16)])
        return c2

      lax.fori_loop(0, CH, mul_row, 0, unroll=False)
      pltpu.sync_copy(hv, agg.at[dv], add=True)

    setA = (sva, tva, dva, sia)
    setB = (svb, tvb, dvb, sib)
    gatA = (hva, rva, sha, sra)
    gatB = (hvb, rvb, shb, srb)

    fire_idx(0, *setA)
    wait_idx(0, *setA)
    fire_gath(setA[0], setA[1], *gatA)
    fire_idx(1, *setB)

    def halfstep(ch, cur, gcur, nxt, gnxt):
      @pl.when(ch + 1 < CPT)
      def _():
        wait_idx(ch + 1, *nxt)
        fire_gath(nxt[0], nxt[1], *gnxt)
      process(cur[0], cur[1], cur[2], *gcur)

      @pl.when(ch + 2 < CPT)
      def _():
        fire_idx(ch + 2, *cur)

    def pair(ch, carry):
      halfstep(ch, setA, gatA, setB, gatB)
      halfstep(ch + 1, setB, gatB, setA, gatA)
      return carry

    lax.fori_loop(0, CPT // 2, lambda i, c: pair(i * 2, c), 0,
                  unroll=False)
    plsc.subcore_barrier()

    pltpu.sync_copy(agg.at[pl.ds(r0, ROWS_PER_TILE)],
                    out_hbm.at[pl.ds(cid * N_PAD + r0, ROWS_PER_TILE)])

  out = body(src, typ, dst, h, rel, zeros)
  return out.reshape(2, N_PAD, BD)


def _relw_body(htT_ref, w1T_ref, b1T_ref, w2T_ref, b2T_ref, emb_ref, o_ref):
  xT = lax.dot_general(w1T_ref[0], htT_ref[...], (((1,), (0,)), ((), ())),
                       preferred_element_type=jnp.float32)
  xT = jnp.maximum(xT + b1T_ref[0], 0.0)
  wT = lax.dot_general(w2T_ref[0], xT, (((1,), (0,)), ((), ())),
                       preferred_element_type=jnp.float32)
  wT = jax.nn.sigmoid(wT + b2T_ref[0])
  e = emb_ref[0]
  o_ref[0] = jnp.concatenate([wT[:, 0:1] * e, wT[:, 1:2] * e], axis=1)


def _rel_tables(htT, w1T, b1T, w2T, b2T, embp):
  return pl.pallas_call(
      _relw_body,
      grid=(N_LAYER,),
      in_specs=[
          pl.BlockSpec((BD, 8), lambda l: (0, 0)),
          pl.BlockSpec((1, 8, BD), lambda l: (l, 0, 0)),
          pl.BlockSpec((1, 8, 1), lambda l: (l, 0, 0)),
          pl.BlockSpec((1, N_RELP, 8), lambda l: (l, 0, 0)),
          pl.BlockSpec((1, N_RELP, 1), lambda l: (l, 0, 0)),
          pl.BlockSpec((1, N_RELP, N_DIM), lambda l: (l, 0, 0)),
      ],
      out_specs=pl.BlockSpec((1, N_RELP, BD), lambda l: (l, 0, 0)),
      out_shape=jax.ShapeDtypeStruct((N_LAYER, N_RELP, BD), jnp.float32),
  )(htT, w1T, b1T, w2T, b2T, embp)


def _tab_body(rel_ref, emb_ref, o_ref):
  e0 = emb_ref[0:1, :]
  e1 = emb_ref[1:2, :]
  z = jnp.zeros((1, N_DIM), jnp.float32)
  hsel = jnp.concatenate([
      jnp.concatenate([z, z], axis=1),
      jnp.concatenate([e0, z], axis=1),
      jnp.concatenate([z, e1], axis=1),
      jnp.concatenate([e0, e1], axis=1),
  ], axis=0)
  rel = rel_ref[...]
  o_ref[...] = (hsel[:, None, :] * rel[None, :, :]).reshape(4 * N_RELP, BD)


def _expand_table(rel0, emb):
  return pl.pallas_call(
      _tab_body,
      out_shape=jax.ShapeDtypeStruct((4 * N_RELP, BD), jnp.float32),
  )(rel0, emb)


def _cls_body(idx_ref, src_ref, typ_ref, o_ref):
  src = src_ref[...]
  cls = (jnp.where(src == idx_ref[0], 1, 0) +
         jnp.where(src == idx_ref[1], 2, 0))
  o_ref[...] = typ_ref[...] + N_RELP * cls


def _cls_types(idx, src_p, typ_p):
  return pl.pallas_call(
      _cls_body,
      in_specs=[
          pl.BlockSpec(memory_space=pltpu.SMEM),
          pl.BlockSpec((N_WORKERS, CPT, CH), lambda: (0, 0, 0)),
          pl.BlockSpec((N_WORKERS, CPT, CH), lambda: (0, 0, 0)),
      ],
      out_specs=pl.BlockSpec((N_WORKERS, CPT, CH), lambda: (0, 0, 0)),
      out_shape=jax.ShapeDtypeStruct((N_WORKERS, CPT, CH), jnp.int32),
  )(idx, src_p, typ_p)


def _init_body(idx_ref, emb_ref, o_ref):
  i = pl.program_id(0)
  rows = jax.lax.broadcasted_iota(jnp.int32, (ROW_BLK, 1), 0) + i * ROW_BLK
  e0 = emb_ref[0:1, :]
  e1 = emb_ref[1:2, :]
  left = jnp.where(rows == idx_ref[0], e0, 0.0)
  right = jnp.where(rows == idx_ref[1], e1, 0.0)
  o_ref[...] = jnp.concatenate([left, right], axis=1)


def _init_hidden(idx, emb):
  return pl.pallas_call(
      _init_body,
      grid=(N_PAD // ROW_BLK,),
      in_specs=[
          pl.BlockSpec(memory_space=pltpu.SMEM),
          pl.BlockSpec((BN, N_DIM), lambda i: (0, 0)),
      ],
      out_specs=pl.BlockSpec((ROW_BLK, BD), lambda i: (i, 0)),
      out_shape=jax.ShapeDtypeStruct((N_PAD, BD), jnp.float32),
  )(idx, emb)


def _lin_body(p_ref, w_ref, b_ref, o_ref):
  a = p_ref[0] + p_ref[1]
  w = w_ref[...]
  b = b_ref[...]
  x1 = lax.dot_general(a[:, :N_DIM], w, (((1,), (0,)), ((), ())),
                       preferred_element_type=jnp.float32)
  x2 = lax.dot_general(a[:, N_DIM:], w, (((1,), (0,)), ((), ())),
                       preferred_element_type=jnp.float32)
  o_ref[...] = jnp.concatenate(
      [jnp.maximum(x1 + b, 0.0), jnp.maximum(x2 + b, 0.0)], axis=1)


def _combine_lin(parts, w, b):
  return pl.pallas_call(
      _lin_body,
      grid=(N_PAD // ROW_BLK,),
      in_specs=[
          pl.BlockSpec((2, ROW_BLK, BD), lambda i: (0, i, 0)),
          pl.BlockSpec((N_DIM, N_DIM), lambda i: (0, 0)),
          pl.BlockSpec((1, N_DIM), lambda i: (0, 0)),
      ],
      out_specs=pl.BlockSpec((ROW_BLK, BD), lambda i: (i, 0)),
      out_shape=jax.ShapeDtypeStruct((N_PAD, BD), jnp.float32),
  )(parts, w, b)


def _score_body(e_ref, w_ref, b_ref, o_ref):
  o_ref[...] = lax.dot_general(e_ref[...], w_ref[...],
                               (((1,), (0,)), ((), ())),
                               preferred_element_type=jnp.float32) + b_ref[...]


def _scores(embp, wp, bp):
  return pl.pallas_call(
      _score_body,
      out_shape=jax.ShapeDtypeStruct((8, 128), jnp.float32),
  )(embp, wp, bp)


@jax.jit
def _run(head, tail, edge_index, edge_type, ent_emb, rel_embs, lin_W,
         lin_b, rel_lin_W, rel_lin_b, attn_W, attn_b, Wr_W, Wr_b):
  dst = edge_index[0].astype(jnp.int32)
  src = edge_index[1].astype(jnp.int32)
  typ = edge_type.astype(jnp.int32)

  npad = E_PADDED - E_EDGES
  src_p = jnp.concatenate([src, jnp.zeros((npad,), jnp.int32)])
  dst_p = jnp.concatenate([dst, jnp.zeros((npad,), jnp.int32)])
  typ_p = jnp.concatenate([typ, jnp.full((npad,), PAD_REL, jnp.int32)])
  src_p = src_p.reshape(N_WORKERS, CPT, CH)
  dst_p = dst_p.reshape(N_WORKERS, CPT, CH)
  typ_p = typ_p.reshape(N_WORKERS, CPT, CH)

  def take2(tbl, ii):
    return jnp.concatenate([
        lax.dynamic_slice(tbl, (ii[0], 0), (1, N_DIM)),
        lax.dynamic_slice(tbl, (ii[1], 0), (1, N_DIM))], axis=0)

  head_embed = take2(ent_emb, head)
  tail_embed = take2(ent_emb, tail)
  ht = jnp.concatenate([head_embed, tail_embed], axis=-1)

  htT = jnp.transpose(ht).reshape(BD, 2)
  htT = jnp.pad(htT, ((0, 0), (0, 6)))
  w1T = jnp.pad(jnp.transpose(rel_lin_W, (0, 2, 1)), ((0, 0), (0, 3), (0, 0)))
  b1T = jnp.pad(rel_lin_b, ((0, 0), (0, 3)))[:, :, None]
  w2T = jnp.pad(jnp.transpose(attn_W, (0, 2, 1)),
                ((0, 0), (0, N_RELP - attn_W.shape[2]), (0, 3)))
  b2T = jnp.pad(attn_b, ((0, 0), (0, N_RELP - attn_b.shape[1])))[:, :, None]
  embp = jnp.pad(rel_embs, ((0, 0), (0, N_RELP - rel_embs.shape[1]), (0, 0)))
  rel_tab = _rel_tables(htT, w1T, b1T, w2T, b2T, embp)

  zeros = jnp.zeros((N_PAD, BD), jnp.float32)
  lin_bb = lin_b.reshape(N_LAYER, 1, N_DIM)
  src_f = src_p.reshape(-1)
  typ_f = typ_p.reshape(-1)
  dst_f = dst_p.reshape(-1)

  def propagate(init_idx, init_emb):
    h = _init_hidden(init_idx.astype(jnp.int32), init_emb)
    for l in range(N_LAYER):
      parts = _rspmm(src_f, typ_f, dst_f, h, rel_tab[l], zeros)
      h = _combine_lin(parts, lin_W[l], lin_bb[l])
    return h

  h_a = propagate(head, head_embed)
  h_b = propagate(tail, tail_embed)

  ht_t = _run_pick(h_a, tail)
  hh_t = _run_pick(h_b, head)

  emb_cat = jnp.concatenate([head_embed, tail_embed, hh_t, ht_t], axis=1)
  embp8 = jnp.pad(emb_cat, ((0, 6), (0, 0)))
  wp = jnp.pad(Wr_W, ((0, 0), (0, 128 - Wr_W.shape[1])))
  bp = jnp.pad(Wr_b, (0, 128 - Wr_b.shape[0])).reshape(1, 128)
  sc = _scores(embp8, wp, bp)
  return sc[:BN, :Wr_W.shape[1]]


def _run_pick(hid, idx):
  r0 = lax.dynamic_slice(hid, (idx[0], 0), (1, N_DIM))
  r1 = lax.dynamic_slice(hid, (idx[1], N_DIM), (1, N_DIM))
  return jnp.concatenate([r0, r1], axis=0)


def kernel(head, tail, edge_index, edge_type, ent_emb, rel_embs, lin_W, lin_b,
           rel_lin_W, rel_lin_b, attn_W, attn_b, Wr_W, Wr_b):
  return _run(head, tail, edge_index, edge_type, ent_emb, rel_embs, lin_W,
              lin_b, rel_lin_W, rel_lin_b, attn_W, attn_b, Wr_W, Wr_b)
